# Initial kernel scaffold; baseline (speedup 1.0000x reference)
import jax
import jax.numpy as jnp
from jax.experimental import pallas as pl


def kernel(x, edge_index, edge_weight, W, b):
    raise NotImplementedError("write your pallas kernel here")


# speedup vs baseline: 4.4756x; 4.4756x over previous
import functools

import jax
import jax.numpy as jnp
from jax import lax
from jax.experimental import pallas as pl
from jax.experimental.pallas import tpu as pltpu
from jax.experimental.pallas import tpu_sc as plsc

N = 10000
E = 320000
D = 128
NC = 2
NS = 16
NW = NC * NS
EPW = E // NW
K = 80
CHUNKS = EPW // K
RPT = 624
REM = N - RPT * NS
P = 104
PP = RPT // P
NF = D // 16


def _sc_partials(x, dst, src, w):
    mesh = plsc.VectorSubcoreMesh(core_axis_name="c", subcore_axis_name="s")

    @functools.partial(
        pl.kernel,
        mesh=mesh,
        out_type=jax.ShapeDtypeStruct((NC, N, D), jnp.float32),
        scratch_types=[
            pltpu.VMEM((K,), jnp.int32),
            pltpu.VMEM((K,), jnp.int32),
            pltpu.VMEM((K,), jnp.float32),
            pltpu.VMEM((K, D), jnp.float32),
            pltpu.VMEM((P, D), jnp.float32),
            pltpu.VMEM_SHARED((N, D), jnp.float32),
            pltpu.SemaphoreType.DMA,
        ],
    )
    def body(x_hbm, dst_hbm, src_hbm, w_hbm, out_hbm,
             srci_v, dsti_v, w_v, rows_v, stage_v, acc_sh, sem):
        cid = lax.axis_index("c")
        sid = lax.axis_index("s")
        wid = cid * NS + sid

        zv = jnp.zeros((16,), jnp.float32)

        def zrow(i, _):
            for f in range(NF):
                stage_v[i, pl.ds(f * 16, 16)] = zv
            return 0

        lax.fori_loop(0, P, zrow, 0)

        def zpiece(p, _):
            pltpu.sync_copy(stage_v, acc_sh.at[pl.ds(sid * RPT + p * P, P)])
            return 0

        lax.fori_loop(0, PP, zpiece, 0)

        @pl.when(sid == NS - 1)
        def _():
            pltpu.sync_copy(stage_v.at[pl.ds(0, REM)],
                            acc_sh.at[pl.ds(NS * RPT, REM)])

        plsc.subcore_barrier()

        def chunk(i, _):
            base = wid * EPW + i * K
            pltpu.sync_copy(src_hbm.at[pl.ds(base, K)], srci_v)
            pltpu.sync_copy(dst_hbm.at[pl.ds(base, K)], dsti_v)
            pltpu.sync_copy(w_hbm.at[pl.ds(base, K)], w_v)
            pltpu.async_copy(x_hbm.at[srci_v], rows_v, sem).wait()

            def group(g, _):
                w16 = w_v[pl.ds(g * 16, 16)]
                for jj in range(16):
                    wj = jnp.full((16,), w16[jj])
                    j = g * 16 + jj
                    for f in range(NF):
                        sl = (j, pl.ds(f * 16, 16))
                        rows_v[sl] = rows_v[sl] * wj
                return 0

            lax.fori_loop(0, K // 16, group, 0)
            pltpu.sync_copy(rows_v, acc_sh.at[dsti_v], add=True)
            return 0

        lax.fori_loop(0, CHUNKS, chunk, 0)
        plsc.subcore_barrier()

        def opiece(p, _):
            base = sid * RPT + p * P
            pltpu.sync_copy(acc_sh.at[pl.ds(base, P)], stage_v)
            pltpu.sync_copy(stage_v, out_hbm.at[cid, pl.ds(base, P)])
            return 0

        lax.fori_loop(0, PP, opiece, 0)

        @pl.when(sid == NS - 1)
        def _():
            pltpu.sync_copy(acc_sh.at[pl.ds(NS * RPT, REM)],
                            stage_v.at[pl.ds(0, REM)])
            pltpu.sync_copy(stage_v.at[pl.ds(0, REM)],
                            out_hbm.at[cid, pl.ds(NS * RPT, REM)])

    return body(x, dst, src, w)


BM = 400


def _tc_finish(partials, W, b2):
    def body(p_ref, w_ref, b_ref, o_ref):
        s = p_ref[0] + p_ref[1]
        o_ref[...] = (
            jnp.dot(s, w_ref[...], preferred_element_type=jnp.float32)
            + b_ref[...]
        )

    return pl.pallas_call(
        body,
        grid=(N // BM,),
        in_specs=[
            pl.BlockSpec((2, BM, D), lambda i: (0, i, 0)),
            pl.BlockSpec((D, D), lambda i: (0, 0)),
            pl.BlockSpec((1, D), lambda i: (0, 0)),
        ],
        out_specs=pl.BlockSpec((BM, D), lambda i: (i, 0)),
        out_shape=jax.ShapeDtypeStruct((N, D), jnp.float32),
    )(partials, W, b2)


def kernel(x, edge_index, edge_weight, W, b):
    dst = edge_index[0]
    src = edge_index[1]
    partials = _sc_partials(x, dst, src, edge_weight)
    return _tc_finish(partials, W, b.reshape(1, D))


# speedup vs baseline: 10.1725x; 2.2729x over previous
import functools

import jax
import jax.numpy as jnp
from jax import lax
from jax.experimental import pallas as pl
from jax.experimental.pallas import tpu as pltpu
from jax.experimental.pallas import tpu_sc as plsc

N = 10000
E = 320000
D = 128
NC = 2
NS = 16
NW = NC * NS
EPW = E // NW
K = 80
CPW = EPW // K
BI = 25
NBLK = CPW // BI
NPAIR = (BI - 1) // 2
RPT = 624
REM = N - RPT * NS
P = 48
PP = RPT // P
NF = D // 16


def _sc_partials(x, dst, src, w):
    mesh = plsc.VectorSubcoreMesh(core_axis_name="c", subcore_axis_name="s")

    @functools.partial(
        pl.kernel,
        mesh=mesh,
        out_type=jax.ShapeDtypeStruct((NC, N, D), jnp.float32),
        scratch_types=[
            pltpu.VMEM((BI * K,), jnp.int32),
            pltpu.VMEM((BI * K,), jnp.int32),
            pltpu.VMEM((BI * K,), jnp.float32),
            pltpu.VMEM((K, D), jnp.float32),
            pltpu.VMEM((K, D), jnp.float32),
            pltpu.VMEM((P, D), jnp.float32),
            pltpu.VMEM_SHARED((N, D), jnp.float32),
            pltpu.SemaphoreType.DMA,
            pltpu.SemaphoreType.DMA,
        ],
    )
    def body(x_hbm, dst_hbm, src_hbm, w_hbm, out_hbm,
             srci_b, dsti_b, w_b, rows0, rows1, stage_v, acc_sh, sem0, sem1):
        cid = lax.axis_index("c")
        sid = lax.axis_index("s")
        wid = cid * NS + sid

        zv = jnp.zeros((16,), jnp.float32)

        def zrow(i, _):
            for f in range(NF):
                stage_v[i, pl.ds(f * 16, 16)] = zv
            return 0

        lax.fori_loop(0, P, zrow, 0)

        def zpiece(p, _):
            pltpu.sync_copy(stage_v, acc_sh.at[pl.ds(sid * RPT + p * P, P)])
            return 0

        lax.fori_loop(0, PP, zpiece, 0)

        @pl.when(sid == NS - 1)
        def _():
            pltpu.sync_copy(stage_v.at[pl.ds(0, REM)],
                            acc_sh.at[pl.ds(NS * RPT, REM)])

        plsc.subcore_barrier()

        def scale(rows, c):
            def grp(g, _):
                w16 = w_b[pl.ds(c * K + g * 16, 16)]
                for jj in range(16):
                    wj = jnp.full((16,), w16[jj])
                    j = g * 16 + jj
                    for f in range(NF):
                        sl = (j, pl.ds(f * 16, 16))
                        rows[sl] = rows[sl] * wj
                return 0

            lax.fori_loop(0, K // 16, grp, 0)

        def gather(rows, sem, c):
            pltpu.async_copy(x_hbm.at[srci_b.at[pl.ds(c * K, K)]], rows, sem)

        def gwait(rows, sem):
            pltpu.make_async_copy(x_hbm.at[pl.ds(0, K)], rows, sem).wait()

        def scatter(rows, c):
            pltpu.sync_copy(rows, acc_sh.at[dsti_b.at[pl.ds(c * K, K)]],
                            add=True)

        def block(bi, _):
            base = wid * EPW + bi * (BI * K)
            pltpu.sync_copy(src_hbm.at[pl.ds(base, BI * K)], srci_b)
            pltpu.sync_copy(dst_hbm.at[pl.ds(base, BI * K)], dsti_b)
            pltpu.sync_copy(w_hbm.at[pl.ds(base, BI * K)], w_b)
            gather(rows0, sem0, 0)
            gather(rows1, sem1, 1)

            def pair(p, _):
                for b in range(2):
                    rows = rows0 if b == 0 else rows1
                    sem = sem0 if b == 0 else sem1
                    c = 2 * p + b
                    gwait(rows, sem)
                    scale(rows, c)
                    scatter(rows, c)

                    @pl.when(c + 2 <= BI - 1)
                    def _():
                        gather(rows, sem, c + 2)

                return 0

            lax.fori_loop(0, NPAIR, pair, 0)
            gwait(rows0, sem0)
            scale(rows0, BI - 1)
            scatter(rows0, BI - 1)
            return 0

        lax.fori_loop(0, NBLK, block, 0)
        plsc.subcore_barrier()

        def opiece(p, _):
            obase = sid * RPT + p * P
            pltpu.sync_copy(acc_sh.at[pl.ds(obase, P)], stage_v)
            pltpu.sync_copy(stage_v, out_hbm.at[cid, pl.ds(obase, P)])
            return 0

        lax.fori_loop(0, PP, opiece, 0)

        @pl.when(sid == NS - 1)
        def _():
            pltpu.sync_copy(acc_sh.at[pl.ds(NS * RPT, REM)],
                            stage_v.at[pl.ds(0, REM)])
            pltpu.sync_copy(stage_v.at[pl.ds(0, REM)],
                            out_hbm.at[cid, pl.ds(NS * RPT, REM)])

    return body(x, dst, src, w)


BM = 400


def _tc_finish(partials, W, b2):
    def body(p_ref, w_ref, b_ref, o_ref):
        s = p_ref[0] + p_ref[1]
        o_ref[...] = (
            jnp.dot(s, w_ref[...], preferred_element_type=jnp.float32)
            + b_ref[...]
        )

    return pl.pallas_call(
        body,
        grid=(N // BM,),
        in_specs=[
            pl.BlockSpec((2, BM, D), lambda i: (0, i, 0)),
            pl.BlockSpec((D, D), lambda i: (0, 0)),
            pl.BlockSpec((1, D), lambda i: (0, 0)),
        ],
        out_specs=pl.BlockSpec((BM, D), lambda i: (i, 0)),
        out_shape=jax.ShapeDtypeStruct((N, D), jnp.float32),
    )(partials, W, b2)


def kernel(x, edge_index, edge_weight, W, b):
    dst = edge_index[0]
    src = edge_index[1]
    partials = _sc_partials(x, dst, src, edge_weight)
    return _tc_finish(partials, W, b.reshape(1, D))
